# Initial kernel scaffold; baseline (speedup 1.0000x reference)
import jax
import jax.numpy as jnp
from jax.experimental import pallas as pl


def kernel(x, edge_index, W1, b1, W2, b2):
    raise NotImplementedError("write your pallas kernel here")


# speedup vs baseline: 27.5891x; 27.5891x over previous
import functools

import jax
import jax.numpy as jnp
from jax import lax
from jax.experimental import pallas as pl
from jax.experimental.pallas import tpu as pltpu
from jax.experimental.pallas import tpu_sc as plsc

NC = 2
NS = 16
L = 16
NW = NC * NS
EB = 80

_MESH = plsc.VectorSubcoreMesh(core_axis_name="c", subcore_axis_name="s")
_SC_PARAMS = pltpu.CompilerParams(use_tc_tiling_on_sc=False,
                                  needs_layout_passes=False)


def _row_offsets(d):
    offs = list(range(0, d - L + 1, L))
    if d % L:
        offs.append(d - L)
    return offs


def _deg_body(dst2d_hbm, out_hbm, idx_v, deg_v):
    cid = lax.axis_index("c")
    sid = lax.axis_index("s")
    wid = sid * NC + cid
    rw = idx_v.shape[0]
    nn = deg_v.shape[0]

    pltpu.sync_copy(dst2d_hbm.at[pl.ds(wid * rw, rw)], idx_v)

    zero16 = jnp.zeros((L,), jnp.float32)

    def zbody(i, carry):
        deg_v[pl.ds(i * L, L)] = zero16
        return carry

    lax.fori_loop(0, nn // L, zbody, 0)

    one16 = jnp.ones((L,), jnp.float32)

    def body(r, carry):
        for c in range(EB // L):
            idx = idx_v[r, pl.ds(c * L, L)]
            plsc.addupdate_scatter(deg_v, [idx], one16)
        return carry

    lax.fori_loop(0, rw, body, 0)
    pltpu.sync_copy(deg_v, out_hbm.at[wid])


def _sc_degree(dst2d, n):
    e_rows = dst2d.shape[0]
    rw = e_rows // NW
    f = pl.kernel(
        _deg_body,
        out_type=jax.ShapeDtypeStruct((NW, n), jnp.float32),
        mesh=_MESH,
        scratch_types=[
            pltpu.VMEM((rw, EB), jnp.int32),
            pltpu.VMEM((n,), jnp.float32),
        ],
        compiler_params=_SC_PARAMS,
    )
    return f(dst2d)


def _agg_body(g_hbm, src2d_hbm, dst2d_hbm, out_hbm,
              sidx_v, didx_v, rows_v, z_v, acc_sh, sem):
    cid = lax.axis_index("c")
    sid = lax.axis_index("s")
    wid = sid * NC + cid
    rw = sidx_v.shape[0]
    d = rows_v.shape[1]
    zr = z_v.shape[0]

    pltpu.sync_copy(src2d_hbm.at[pl.ds(wid * rw, rw)], sidx_v)
    pltpu.sync_copy(dst2d_hbm.at[pl.ds(wid * rw, rw)], didx_v)

    zero16 = jnp.zeros((L,), jnp.float32)
    offs = _row_offsets(d)

    def zbody(i, carry):
        for off in offs:
            z_v[i, pl.ds(off, L)] = zero16
        return carry

    lax.fori_loop(0, zr, zbody, 0)
    pltpu.sync_copy(z_v, acc_sh.at[pl.ds(sid * zr, zr)])
    plsc.subcore_barrier()

    def body(r, carry):
        pltpu.async_copy(g_hbm.at[sidx_v.at[r]], rows_v, sem).wait()
        pltpu.sync_copy(rows_v, acc_sh.at[didx_v.at[r]], add=True)
        return carry

    lax.fori_loop(0, rw, body, 0)
    plsc.subcore_barrier()
    pltpu.sync_copy(acc_sh.at[pl.ds(sid * zr, zr)],
                    out_hbm.at[cid, pl.ds(sid * zr, zr)])


def _sc_aggregate(g, src2d, dst2d):
    n, d = g.shape
    rw = src2d.shape[0] // NW
    f = pl.kernel(
        _agg_body,
        out_type=jax.ShapeDtypeStruct((NC, n, d), jnp.float32),
        mesh=_MESH,
        scratch_types=[
            pltpu.VMEM((rw, EB), jnp.int32),
            pltpu.VMEM((rw, EB), jnp.int32),
            pltpu.VMEM((EB, d), jnp.float32),
            pltpu.VMEM((n // NS, d), jnp.float32),
            pltpu.VMEM_SHARED((n, d), jnp.float32),
            pltpu.SemaphoreType.DMA,
        ],
        compiler_params=_SC_PARAMS,
    )
    return f(g, src2d, dst2d)


def _tc1_body(x_ref, w1_ref, dpt_ref, g1_ref, dinv_ref):
    deg = jnp.sum(dpt_ref[...], axis=1, keepdims=True) + 1.0
    dinv = lax.rsqrt(deg)
    h = jnp.dot(x_ref[...], w1_ref[...], preferred_element_type=jnp.float32)
    g1_ref[...] = h * dinv
    dinv_ref[...] = dinv


def _tc2_body(a_ref, g1_ref, dinv_ref, b1_ref, w2_ref, g2_ref):
    dinv = dinv_ref[...]
    z = dinv * (a_ref[0] + a_ref[1] + g1_ref[...]) + b1_ref[...]
    r = jnp.maximum(z, 0.0)
    h2 = jnp.dot(r, w2_ref[...], preferred_element_type=jnp.float32)
    g2_ref[...] = h2 * dinv


def _tc3_body(a_ref, g2_ref, dinv_ref, b2_ref, o_ref):
    z = dinv_ref[...] * (a_ref[0] + a_ref[1] + g2_ref[...]) + b2_ref[...]
    m = jnp.max(z, axis=1, keepdims=True)
    lse = jnp.log(jnp.sum(jnp.exp(z - m), axis=1, keepdims=True))
    o_ref[...] = z - m - lse


def _tc1(x, w1, dpt):
    n = x.shape[0]
    h = w1.shape[1]
    return pl.pallas_call(
        _tc1_body,
        out_shape=(jax.ShapeDtypeStruct((n, h), jnp.float32),
                   jax.ShapeDtypeStruct((n, 1), jnp.float32)),
    )(x, w1, dpt)


def _tc2(a1, g1, dinv, b1, w2):
    n = g1.shape[0]
    c = w2.shape[1]
    return pl.pallas_call(
        _tc2_body,
        out_shape=jax.ShapeDtypeStruct((n, c), jnp.float32),
    )(a1, g1, dinv, b1, w2)


def _tc3(a2, g2, dinv, b2):
    return pl.pallas_call(
        _tc3_body,
        out_shape=jax.ShapeDtypeStruct(g2.shape, jnp.float32),
    )(a2, g2, dinv, b2)


def kernel(x, edge_index, W1, b1, W2, b2):
    n = x.shape[0]
    e = edge_index.shape[1]
    src2d = edge_index[0].reshape(e // EB, EB)
    dst2d = edge_index[1].reshape(e // EB, EB)

    dp = _sc_degree(dst2d, n)
    g1, dinv = _tc1(x, W1, dp.T)
    a1 = _sc_aggregate(g1, src2d, dst2d)
    g2 = _tc2(a1, g1, dinv, b1, W2)
    a2 = _sc_aggregate(g2, src2d, dst2d)
    return _tc3(a2, g2, dinv, b2)


# speedup vs baseline: 50.3739x; 1.8259x over previous
import functools

import jax
import jax.numpy as jnp
from jax import lax
from jax.experimental import pallas as pl
from jax.experimental.pallas import tpu as pltpu
from jax.experimental.pallas import tpu_sc as plsc

NC = 2
NS = 16
L = 16
NW = NC * NS
EB = 80

_MESH = plsc.VectorSubcoreMesh(core_axis_name="c", subcore_axis_name="s")
_SC_PARAMS = pltpu.CompilerParams(use_tc_tiling_on_sc=False,
                                  needs_layout_passes=False)


def _row_offsets(d):
    offs = list(range(0, d - L + 1, L))
    if d % L:
        offs.append(d - L)
    return offs


def _deg_body(dst2d_hbm, out_hbm, idx_v, deg_v):
    cid = lax.axis_index("c")
    sid = lax.axis_index("s")
    wid = sid * NC + cid
    rw = idx_v.shape[0]
    nn = deg_v.shape[0]

    pltpu.sync_copy(dst2d_hbm.at[pl.ds(wid * rw, rw)], idx_v)

    zero16 = jnp.zeros((L,), jnp.float32)

    def zbody(i, carry):
        deg_v[pl.ds(i * L, L)] = zero16
        return carry

    lax.fori_loop(0, nn // L, zbody, 0)

    one16 = jnp.ones((L,), jnp.float32)

    def body(r, carry):
        for c in range(EB // L):
            idx = idx_v[r, pl.ds(c * L, L)]
            plsc.addupdate_scatter(deg_v, [idx], one16)
        return carry

    lax.fori_loop(0, rw, body, 0)
    pltpu.sync_copy(deg_v, out_hbm.at[wid])


def _sc_degree(dst2d, n):
    e_rows = dst2d.shape[0]
    rw = e_rows // NW
    f = pl.kernel(
        _deg_body,
        out_type=jax.ShapeDtypeStruct((NW, n), jnp.float32),
        mesh=_MESH,
        scratch_types=[
            pltpu.VMEM((rw, EB), jnp.int32),
            pltpu.VMEM((n,), jnp.float32),
        ],
        compiler_params=_SC_PARAMS,
    )
    return f(dst2d)


NBUF = 5


def _agg_body(g_hbm, src2d_hbm, dst2d_hbm, out_hbm,
              sidx_v, didx_v, rows_v, z_v, acc_sh, gsem, ssem):
    cid = lax.axis_index("c")
    sid = lax.axis_index("s")
    wid = sid * NC + cid
    rw = sidx_v.shape[0]
    d = rows_v.shape[2]
    zr = z_v.shape[0]
    nblk = rw // NBUF

    pltpu.sync_copy(src2d_hbm.at[pl.ds(wid * rw, rw)], sidx_v)
    pltpu.sync_copy(dst2d_hbm.at[pl.ds(wid * rw, rw)], didx_v)

    zero16 = jnp.zeros((L,), jnp.float32)
    offs = _row_offsets(d)

    def zbody(i, carry):
        for off in offs:
            z_v[i, pl.ds(off, L)] = zero16
        return carry

    lax.fori_loop(0, zr, zbody, 0)
    pltpu.sync_copy(z_v, acc_sh.at[pl.ds(sid * zr, zr)])
    plsc.subcore_barrier()

    def gather(r, b):
        return pltpu.async_copy(g_hbm.at[sidx_v.at[r]], rows_v.at[b],
                                gsem.at[b])

    def scatter(r, b):
        return pltpu.async_copy(rows_v.at[b], acc_sh.at[didx_v.at[r]],
                                ssem.at[b], add=True)

    def wait_gather(r, b):
        pltpu.make_async_copy(g_hbm.at[sidx_v.at[r]], rows_v.at[b],
                              gsem.at[b]).wait()

    def wait_scatter(r, b):
        pltpu.make_async_copy(rows_v.at[b], acc_sh.at[didx_v.at[r]],
                              ssem.at[b]).wait()

    for b in range(NBUF):
        gather(b, b)

    def blk(i, carry):
        base = i * NBUF
        for b in range(NBUF):
            wait_gather(base + b, b)
            scatter(base + b, b)

        @pl.when(i < nblk - 1)
        def _():
            for b in range(NBUF):
                wait_scatter(base + b, b)
                gather(base + NBUF + b, b)

        @pl.when(i == nblk - 1)
        def _():
            for b in range(NBUF):
                wait_scatter(base + b, b)

        return carry

    lax.fori_loop(0, nblk, blk, 0)
    plsc.subcore_barrier()
    pltpu.sync_copy(acc_sh.at[pl.ds(sid * zr, zr)],
                    out_hbm.at[cid, pl.ds(sid * zr, zr)])


def _sc_aggregate(g, src2d, dst2d):
    n, d = g.shape
    rw = src2d.shape[0] // NW
    f = pl.kernel(
        _agg_body,
        out_type=jax.ShapeDtypeStruct((NC, n, d), jnp.float32),
        mesh=_MESH,
        scratch_types=[
            pltpu.VMEM((rw, EB), jnp.int32),
            pltpu.VMEM((rw, EB), jnp.int32),
            pltpu.VMEM((NBUF, EB, d), jnp.float32),
            pltpu.VMEM((n // NS, d), jnp.float32),
            pltpu.VMEM_SHARED((n, d), jnp.float32),
            pltpu.SemaphoreType.DMA((NBUF,)),
            pltpu.SemaphoreType.DMA((NBUF,)),
        ],
        compiler_params=_SC_PARAMS,
    )
    return f(g, src2d, dst2d)


def _tc1_body(x_ref, w1_ref, dpt_ref, g1_ref, dinv_ref):
    deg = jnp.sum(dpt_ref[...], axis=1, keepdims=True) + 1.0
    dinv = lax.rsqrt(deg)
    h = jnp.dot(x_ref[...], w1_ref[...], preferred_element_type=jnp.float32)
    g1_ref[...] = h * dinv
    dinv_ref[...] = dinv


def _tc2_body(a_ref, g1_ref, dinv_ref, b1_ref, w2_ref, g2_ref):
    dinv = dinv_ref[...]
    z = dinv * (a_ref[0] + a_ref[1] + g1_ref[...]) + b1_ref[...]
    r = jnp.maximum(z, 0.0)
    h2 = jnp.dot(r, w2_ref[...], preferred_element_type=jnp.float32)
    g2_ref[...] = h2 * dinv


def _tc3_body(a_ref, g2_ref, dinv_ref, b2_ref, o_ref):
    z = dinv_ref[...] * (a_ref[0] + a_ref[1] + g2_ref[...]) + b2_ref[...]
    m = jnp.max(z, axis=1, keepdims=True)
    lse = jnp.log(jnp.sum(jnp.exp(z - m), axis=1, keepdims=True))
    o_ref[...] = z - m - lse


def _tc1(x, w1, dpt):
    n = x.shape[0]
    h = w1.shape[1]
    return pl.pallas_call(
        _tc1_body,
        out_shape=(jax.ShapeDtypeStruct((n, h), jnp.float32),
                   jax.ShapeDtypeStruct((n, 1), jnp.float32)),
    )(x, w1, dpt)


def _tc2(a1, g1, dinv, b1, w2):
    n = g1.shape[0]
    c = w2.shape[1]
    return pl.pallas_call(
        _tc2_body,
        out_shape=jax.ShapeDtypeStruct((n, c), jnp.float32),
    )(a1, g1, dinv, b1, w2)


def _tc3(a2, g2, dinv, b2):
    return pl.pallas_call(
        _tc3_body,
        out_shape=jax.ShapeDtypeStruct(g2.shape, jnp.float32),
    )(a2, g2, dinv, b2)


def kernel(x, edge_index, W1, b1, W2, b2):
    n = x.shape[0]
    e = edge_index.shape[1]
    src2d = edge_index[0].reshape(e // EB, EB)
    dst2d = edge_index[1].reshape(e // EB, EB)

    dp = _sc_degree(dst2d, n)
    g1, dinv = _tc1(x, W1, dp.T)
    a1 = _sc_aggregate(g1, src2d, dst2d)
    g2 = _tc2(a1, g1, dinv, b1, W2)
    a2 = _sc_aggregate(g2, src2d, dst2d)
    return _tc3(a2, g2, dinv, b2)


# speedup vs baseline: 53.3665x; 1.0594x over previous
import functools

import jax
import jax.numpy as jnp
from jax import lax
from jax.experimental import pallas as pl
from jax.experimental.pallas import tpu as pltpu
from jax.experimental.pallas import tpu_sc as plsc

NC = 2
NS = 16
L = 16
NW = NC * NS
EB = 80

_MESH = plsc.VectorSubcoreMesh(core_axis_name="c", subcore_axis_name="s")
_SC_PARAMS = pltpu.CompilerParams(use_tc_tiling_on_sc=False,
                                  needs_layout_passes=False)


def _row_offsets(d):
    offs = list(range(0, d - L + 1, L))
    if d % L:
        offs.append(d - L)
    return offs


def _deg_body(e3_hbm, out_hbm, idx_v, hist_v, iidx_v, acc_sh):
    cid = lax.axis_index("c")
    sid = lax.axis_index("s")
    wid = sid * NC + cid
    rw = idx_v.shape[0]
    hr = hist_v.shape[0]
    tr = hr // NS

    pltpu.sync_copy(e3_hbm.at[1, pl.ds(wid * rw, rw)], idx_v)

    zero16 = jnp.zeros((L,), jnp.float32)
    iota16 = lax.iota(jnp.int32, L)

    def zbody(i, carry):
        hist_v[i, :] = zero16
        return carry

    lax.fori_loop(0, hr, zbody, 0)

    for b in range(hr // EB):
        for c in range(EB // L):
            iidx_v[b, pl.ds(c * L, L)] = b * EB + c * L + iota16
    pltpu.sync_copy(hist_v.at[pl.ds(sid * tr, tr)],
                    acc_sh.at[pl.ds(sid * tr, tr)])
    plsc.subcore_barrier()

    one16 = jnp.ones((L,), jnp.float32)
    m15 = jnp.full((L,), L - 1, jnp.int32)

    def body(r, carry):
        for c in range(EB // L):
            idx = idx_v[r, pl.ds(c * L, L)]
            row = lax.shift_right_logical(idx, 4)
            col = jnp.bitwise_and(idx, m15)
            plsc.addupdate_scatter(hist_v, [row, col], one16)
        return carry

    lax.fori_loop(0, rw, body, 0)

    for b in range(hr // EB):
        pltpu.sync_copy(hist_v.at[pl.ds(b * EB, EB)],
                        acc_sh.at[iidx_v.at[b]], add=True)
    plsc.subcore_barrier()
    pltpu.sync_copy(acc_sh.at[pl.ds(sid * tr, tr)],
                    out_hbm.at[cid, pl.ds(sid * tr, tr)])


def _sc_degree(e3, n_pad_rows):
    rw = e3.shape[1] // NW
    f = pl.kernel(
        _deg_body,
        out_type=jax.ShapeDtypeStruct((NC, n_pad_rows, L), jnp.float32),
        mesh=_MESH,
        scratch_types=[
            pltpu.VMEM((rw, EB), jnp.int32),
            pltpu.VMEM((n_pad_rows, L), jnp.float32),
            pltpu.VMEM((n_pad_rows // EB, EB), jnp.int32),
            pltpu.VMEM_SHARED((n_pad_rows, L), jnp.float32),
        ],
        compiler_params=_SC_PARAMS,
    )
    return f(e3)


NBUF = 5


def _agg_body(g_hbm, e3_hbm, out_hbm,
              sidx_v, didx_v, rows_v, z_v, acc_sh, gsem, ssem):
    cid = lax.axis_index("c")
    sid = lax.axis_index("s")
    wid = sid * NC + cid
    rw = sidx_v.shape[0]
    d = rows_v.shape[2]
    zr = z_v.shape[0]
    nblk = rw // NBUF

    pltpu.sync_copy(e3_hbm.at[0, pl.ds(wid * rw, rw)], sidx_v)
    pltpu.sync_copy(e3_hbm.at[1, pl.ds(wid * rw, rw)], didx_v)

    zero16 = jnp.zeros((L,), jnp.float32)
    offs = _row_offsets(d)

    def zbody(i, carry):
        for off in offs:
            z_v[i, pl.ds(off, L)] = zero16
        return carry

    lax.fori_loop(0, zr, zbody, 0)
    pltpu.sync_copy(z_v, acc_sh.at[pl.ds(sid * zr, zr)])
    plsc.subcore_barrier()

    def gather(r, b):
        return pltpu.async_copy(g_hbm.at[sidx_v.at[r]], rows_v.at[b],
                                gsem.at[b])

    def scatter(r, b):
        return pltpu.async_copy(rows_v.at[b], acc_sh.at[didx_v.at[r]],
                                ssem.at[b], add=True)

    def wait_gather(r, b):
        pltpu.make_async_copy(g_hbm.at[sidx_v.at[r]], rows_v.at[b],
                              gsem.at[b]).wait()

    def wait_scatter(r, b):
        pltpu.make_async_copy(rows_v.at[b], acc_sh.at[didx_v.at[r]],
                              ssem.at[b]).wait()

    for b in range(NBUF):
        gather(b, b)

    def blk(i, carry):
        base = i * NBUF
        for b in range(NBUF):
            wait_gather(base + b, b)
            scatter(base + b, b)

        @pl.when(i < nblk - 1)
        def _():
            for b in range(NBUF):
                wait_scatter(base + b, b)
                gather(base + NBUF + b, b)

        @pl.when(i == nblk - 1)
        def _():
            for b in range(NBUF):
                wait_scatter(base + b, b)

        return carry

    lax.fori_loop(0, nblk, blk, 0)
    plsc.subcore_barrier()
    pltpu.sync_copy(acc_sh.at[pl.ds(sid * zr, zr)],
                    out_hbm.at[cid, pl.ds(sid * zr, zr)])


def _sc_aggregate(g, e3):
    n, d = g.shape
    rw = e3.shape[1] // NW
    f = pl.kernel(
        _agg_body,
        out_type=jax.ShapeDtypeStruct((NC, n, d), jnp.float32),
        mesh=_MESH,
        scratch_types=[
            pltpu.VMEM((rw, EB), jnp.int32),
            pltpu.VMEM((rw, EB), jnp.int32),
            pltpu.VMEM((NBUF, EB, d), jnp.float32),
            pltpu.VMEM((n // NS, d), jnp.float32),
            pltpu.VMEM_SHARED((n, d), jnp.float32),
            pltpu.SemaphoreType.DMA((NBUF,)),
            pltpu.SemaphoreType.DMA((NBUF,)),
        ],
        compiler_params=_SC_PARAMS,
    )
    return f(g, e3)


def _tc1_body(x_ref, w1_ref, dinv_ref, g1_ref):
    h = jnp.dot(x_ref[...], w1_ref[...], preferred_element_type=jnp.float32)
    g1_ref[...] = h * dinv_ref[...]


def _tc2_body(a_ref, g1_ref, dinv_ref, b1_ref, w2_ref, g2_ref):
    dinv = dinv_ref[...]
    z = dinv * (a_ref[0] + a_ref[1] + g1_ref[...]) + b1_ref[...]
    r = jnp.maximum(z, 0.0)
    h2 = jnp.dot(r, w2_ref[...], preferred_element_type=jnp.float32)
    g2_ref[...] = h2 * dinv


def _tc3_body(a_ref, g2_ref, dinv_ref, b2_ref, o_ref):
    z = dinv_ref[...] * (a_ref[0] + a_ref[1] + g2_ref[...]) + b2_ref[...]
    m = jnp.max(z, axis=1, keepdims=True)
    lse = jnp.log(jnp.sum(jnp.exp(z - m), axis=1, keepdims=True))
    o_ref[...] = z - m - lse


def _tc1(x, w1, dinv):
    n = x.shape[0]
    h = w1.shape[1]
    return pl.pallas_call(
        _tc1_body,
        out_shape=jax.ShapeDtypeStruct((n, h), jnp.float32),
    )(x, w1, dinv)


def _tc2(a1, g1, dinv, b1, w2):
    n = g1.shape[0]
    c = w2.shape[1]
    return pl.pallas_call(
        _tc2_body,
        out_shape=jax.ShapeDtypeStruct((n, c), jnp.float32),
    )(a1, g1, dinv, b1, w2)


def _tc3(a2, g2, dinv, b2):
    return pl.pallas_call(
        _tc3_body,
        out_shape=jax.ShapeDtypeStruct(g2.shape, jnp.float32),
    )(a2, g2, dinv, b2)


def kernel(x, edge_index, W1, b1, W2, b2):
    n = x.shape[0]
    e = edge_index.shape[1]
    e3 = edge_index.reshape(2, e // EB, EB)
    n_rows = -(-n // (L * NS)) * NS

    dp = _sc_degree(e3, n_rows)
    deg = (dp[0] + dp[1]).reshape(n_rows * L)[:n] + 1.0
    dinv = lax.rsqrt(deg)[:, None]
    g1 = _tc1(x, W1, dinv)
    a1 = _sc_aggregate(g1, e3)
    g2 = _tc2(a1, g1, dinv, b1, W2)
    a2 = _sc_aggregate(g2, e3)
    return _tc3(a2, g2, dinv, b2)


# speedup vs baseline: 53.9714x; 1.0113x over previous
import functools
import math

import jax
import jax.numpy as jnp
from jax import lax
from jax.experimental import pallas as pl
from jax.experimental.pallas import tpu as pltpu
from jax.experimental.pallas import tpu_sc as plsc

NC = 2
NS = 16
L = 16
NW = NC * NS
EB = 80

_MESH = plsc.VectorSubcoreMesh(core_axis_name="c", subcore_axis_name="s",
                               num_cores=NC, num_subcores=NS)
_SC_PARAMS = pltpu.CompilerParams(use_tc_tiling_on_sc=False,
                                  needs_layout_passes=False)


def _row_offsets(d):
    offs = list(range(0, d - L + 1, L))
    if d % L:
        offs.append(d - L)
    return offs


def _deg_body(e3_hbm, out_hbm, idx_v, hist_v, iidx_v, acc_sh):
    cid = lax.axis_index("c")
    sid = lax.axis_index("s")
    wid = sid * NC + cid
    rw = idx_v.shape[0]
    hr = hist_v.shape[0]
    tr = hr // NS

    pltpu.sync_copy(e3_hbm.at[1, pl.ds(wid * rw, rw)], idx_v)

    zero16 = jnp.zeros((L,), jnp.float32)
    iota16 = lax.iota(jnp.int32, L)

    def zbody(i, carry):
        hist_v[i, :] = zero16
        return carry

    lax.fori_loop(0, hr, zbody, 0)

    for b in range(hr // EB):
        for c in range(EB // L):
            iidx_v[b, pl.ds(c * L, L)] = b * EB + c * L + iota16
    pltpu.sync_copy(hist_v.at[pl.ds(sid * tr, tr)],
                    acc_sh.at[pl.ds(sid * tr, tr)])
    plsc.subcore_barrier()

    one16 = jnp.ones((L,), jnp.float32)
    m15 = jnp.full((L,), L - 1, jnp.int32)

    def body(r, carry):
        for c in range(EB // L):
            idx = idx_v[r, pl.ds(c * L, L)]
            row = lax.shift_right_logical(idx, 4)
            col = jnp.bitwise_and(idx, m15)
            plsc.addupdate_scatter(hist_v, [row, col], one16)
        return carry

    lax.fori_loop(0, rw, body, 0)

    for b in range(hr // EB):
        pltpu.sync_copy(hist_v.at[pl.ds(b * EB, EB)],
                        acc_sh.at[iidx_v.at[b]], add=True)
    plsc.subcore_barrier()
    pltpu.sync_copy(acc_sh.at[pl.ds(sid * tr, tr)],
                    out_hbm.at[cid, pl.ds(sid * tr, tr)])


def _sc_degree(e3, n_pad_rows):
    rw = e3.shape[1] // NW
    f = pl.kernel(
        _deg_body,
        out_type=jax.ShapeDtypeStruct((NC, n_pad_rows, L), jnp.float32),
        mesh=_MESH,
        scratch_types=[
            pltpu.VMEM((rw, EB), jnp.int32),
            pltpu.VMEM((n_pad_rows, L), jnp.float32),
            pltpu.VMEM((n_pad_rows // EB, EB), jnp.int32),
            pltpu.VMEM_SHARED((n_pad_rows, L), jnp.float32),
        ],
        compiler_params=_SC_PARAMS,
    )
    return f(e3)


NBUF = 5


def _agg_body(g_hbm, e3_hbm, out_hbm,
              sidx_v, didx_v, rows_v, z_v, acc_sh, gsem, ssem):
    cid = lax.axis_index("c")
    sid = lax.axis_index("s")
    wid = sid * NC + cid
    rw = sidx_v.shape[0]
    d = rows_v.shape[2]
    zr = z_v.shape[0]
    nblk = rw // NBUF

    pltpu.sync_copy(e3_hbm.at[0, pl.ds(wid * rw, rw)], sidx_v)
    pltpu.sync_copy(e3_hbm.at[1, pl.ds(wid * rw, rw)], didx_v)

    zero16 = jnp.zeros((L,), jnp.float32)
    offs = _row_offsets(d)

    def zbody(i, carry):
        for off in offs:
            z_v[i, pl.ds(off, L)] = zero16
        return carry

    lax.fori_loop(0, zr, zbody, 0)
    pltpu.sync_copy(z_v, acc_sh.at[pl.ds(sid * zr, zr)])
    plsc.subcore_barrier()

    def gather(r, b):
        return pltpu.async_copy(g_hbm.at[sidx_v.at[r]], rows_v.at[b],
                                gsem.at[b])

    def scatter(r, b):
        return pltpu.async_copy(rows_v.at[b], acc_sh.at[didx_v.at[r]],
                                ssem.at[b], add=True)

    def wait_gather(r, b):
        pltpu.make_async_copy(g_hbm.at[sidx_v.at[r]], rows_v.at[b],
                              gsem.at[b]).wait()

    def wait_scatter(r, b):
        pltpu.make_async_copy(rows_v.at[b], acc_sh.at[didx_v.at[r]],
                              ssem.at[b]).wait()

    for b in range(NBUF):
        gather(b, b)

    def blk(i, carry):
        base = i * NBUF
        for b in range(NBUF):
            wait_gather(base + b, b)
            scatter(base + b, b)

        @pl.when(i < nblk - 1)
        def _():
            for b in range(NBUF):
                wait_scatter(base + b, b)
                gather(base + NBUF + b, b)

        @pl.when(i == nblk - 1)
        def _():
            for b in range(NBUF):
                wait_scatter(base + b, b)

        return carry

    lax.fori_loop(0, nblk, blk, 0)
    plsc.subcore_barrier()
    pltpu.sync_copy(acc_sh.at[pl.ds(sid * zr, zr)],
                    out_hbm.at[cid, pl.ds(sid * zr, zr)])


def _sc_aggregate(g, e3):
    n, d = g.shape
    rw = e3.shape[1] // NW
    f = pl.kernel(
        _agg_body,
        out_type=jax.ShapeDtypeStruct((NC, n, d), jnp.float32),
        mesh=_MESH,
        scratch_types=[
            pltpu.VMEM((rw, EB), jnp.int32),
            pltpu.VMEM((rw, EB), jnp.int32),
            pltpu.VMEM((NBUF, EB, d), jnp.float32),
            pltpu.VMEM((n // NS, d), jnp.float32),
            pltpu.VMEM_SHARED((n, d), jnp.float32),
            pltpu.SemaphoreType.DMA((NBUF,)),
            pltpu.SemaphoreType.DMA((NBUF,)),
        ],
        compiler_params=_SC_PARAMS,
    )
    return f(g, e3)


def _pack_rows(z, d):
    n = z.shape[0]
    lcm = d * 128 // math.gcd(d, 128)
    p, r = lcm // d, lcm // 128
    z3 = jnp.reshape(z, (n // p, p, d))
    rows = []
    for i in range(r):
        pieces, start = [], 128 * i
        while start < 128 * (i + 1):
            k, o = start // d, start % d
            end = min(d, o + 128 * (i + 1) - start)
            pieces.append(z3[:, k, o:end])
            start += end - o
        rows.append(jnp.concatenate(pieces, axis=1))
    st = jnp.stack(rows, axis=1)
    return jnp.reshape(st, (n * d // 128, 128))


def _unpack_rows(zq, n, d):
    lcm = d * 128 // math.gcd(d, 128)
    p, r = lcm // d, lcm // 128
    z3 = jnp.reshape(zq, (n // p, r, 128))
    nodes = []
    for k in range(p):
        pieces, start = [], d * k
        while start < d * (k + 1):
            i, o = start // 128, start % 128
            end = min(128, o + d * (k + 1) - start)
            pieces.append(z3[:, i, o:end])
            start += end - o
        nodes.append(jnp.concatenate(pieces, axis=1))
    st = jnp.stack(nodes, axis=1)
    return jnp.reshape(st, (n, d))


def _tc1_body(x_ref, w1_ref, dinv16_ref, g1p_ref):
    h = jnp.dot(x_ref[...], w1_ref[...], preferred_element_type=jnp.float32)
    g1p_ref[...] = _pack_rows(h, w1_ref.shape[1]) * dinv16_ref[...]


def _tc2_body(a_ref, g1p_ref, dinv16_ref, dinv40_ref, b1r_ref, w2_ref,
              g2p_ref):
    hdim, c = w2_ref.shape
    n = a_ref.shape[1] * 128 // hdim
    zp = dinv16_ref[...] * (a_ref[0] + a_ref[1] + g1p_ref[...]) + b1r_ref[...]
    rs = dinv16_ref[...] * jnp.maximum(zp, 0.0)
    r = _unpack_rows(rs, n, hdim)
    h2 = jnp.dot(r, w2_ref[...], preferred_element_type=jnp.float32)
    g2p_ref[...] = _pack_rows(h2, c)


def _tc3_body(a_ref, g2p_ref, dinv40_ref, b2_ref, o_ref):
    n, c = o_ref.shape
    zp = dinv40_ref[...] * (a_ref[0] + a_ref[1] + g2p_ref[...])
    z = _unpack_rows(zp, n, c) + b2_ref[...]
    m = jnp.max(z, axis=1, keepdims=True)
    lse = jnp.log(jnp.sum(jnp.exp(z - m), axis=1, keepdims=True))
    o_ref[...] = z - m - lse


def _tc1(x, w1, dinv16p):
    return pl.pallas_call(
        _tc1_body,
        out_shape=jax.ShapeDtypeStruct(dinv16p.shape, jnp.float32),
    )(x, w1, dinv16p)


def _tc2(a1p, g1p, dinv16p, dinv40p, b1r, w2):
    return pl.pallas_call(
        _tc2_body,
        out_shape=jax.ShapeDtypeStruct(dinv40p.shape, jnp.float32),
    )(a1p, g1p, dinv16p, dinv40p, b1r, w2)


def _tc3(a2p, g2p, dinv40p, b2, n):
    c = b2.shape[0]
    return pl.pallas_call(
        _tc3_body,
        out_shape=jax.ShapeDtypeStruct((n, c), jnp.float32),
    )(a2p, g2p, dinv40p, b2)


def kernel(x, edge_index, W1, b1, W2, b2):
    n = x.shape[0]
    e = edge_index.shape[1]
    e3 = edge_index.reshape(2, e // EB, EB)
    n_rows = -(-n // (L * NS)) * NS

    hdim = W1.shape[1]
    c = W2.shape[1]

    dp = _sc_degree(e3, n_rows)
    deg = (dp[0] + dp[1]).reshape(n_rows * L)[:n] + 1.0
    dinv = lax.rsqrt(deg)
    dinv16p = jnp.repeat(dinv, hdim).reshape(n * hdim // 128, 128)
    dinv40p = jnp.repeat(dinv, c).reshape(n * c // 128, 128)
    b1r = jnp.tile(b1, 128 // hdim)

    g1p = _tc1(x, W1, dinv16p)
    g1 = g1p.reshape(n, hdim)
    a1 = _sc_aggregate(g1, e3)
    a1p = a1.reshape(NC, n * hdim // 128, 128)
    g2p = _tc2(a1p, g1p, dinv16p, dinv40p, b1r, W2)
    a2 = _sc_aggregate(g2p.reshape(n, c), e3)
    a2p = a2.reshape(NC, n * c // 128, 128)
    return _tc3(a2p, g2p, dinv40p, b2, n)


# speedup vs baseline: 63.2402x; 1.1717x over previous
import functools
import math

import jax
import jax.numpy as jnp
from jax import lax
from jax.experimental import pallas as pl
from jax.experimental.pallas import tpu as pltpu
from jax.experimental.pallas import tpu_sc as plsc

NC = 2
NS = 16
L = 16
NW = NC * NS
EB = 80

_MESH = plsc.VectorSubcoreMesh(core_axis_name="c", subcore_axis_name="s",
                               num_cores=NC, num_subcores=NS)
_SC_PARAMS = pltpu.CompilerParams(use_tc_tiling_on_sc=False,
                                  needs_layout_passes=False)


def _row_offsets(d):
    offs = list(range(0, d - L + 1, L))
    if d % L:
        offs.append(d - L)
    return offs


def _deg_body(e3_hbm, out_hbm, idx_v, hist_v, iidx_v, acc_sh):
    cid = lax.axis_index("c")
    sid = lax.axis_index("s")
    wid = sid * NC + cid
    rw = idx_v.shape[0]
    hr = hist_v.shape[0]
    tr = hr // NS

    pltpu.sync_copy(e3_hbm.at[1, pl.ds(wid * rw, rw)], idx_v)

    zero16 = jnp.zeros((L,), jnp.float32)
    iota16 = lax.iota(jnp.int32, L)

    def zbody(i, carry):
        hist_v[i, :] = zero16
        return carry

    lax.fori_loop(0, hr, zbody, 0)

    for b in range(hr // EB):
        for c in range(EB // L):
            iidx_v[b, pl.ds(c * L, L)] = b * EB + c * L + iota16
    pltpu.sync_copy(hist_v.at[pl.ds(sid * tr, tr)],
                    acc_sh.at[pl.ds(sid * tr, tr)])
    plsc.subcore_barrier()

    one16 = jnp.ones((L,), jnp.float32)
    m15 = jnp.full((L,), L - 1, jnp.int32)

    def body(r, carry):
        for c in range(EB // L):
            idx = idx_v[r, pl.ds(c * L, L)]
            row = lax.shift_right_logical(idx, 4)
            col = jnp.bitwise_and(idx, m15)
            plsc.addupdate_scatter(hist_v, [row, col], one16)
        return carry

    lax.fori_loop(0, rw, body, 0)

    for b in range(hr // EB):
        pltpu.sync_copy(hist_v.at[pl.ds(b * EB, EB)],
                        acc_sh.at[iidx_v.at[b]], add=True)
    plsc.subcore_barrier()
    pltpu.sync_copy(acc_sh.at[pl.ds(sid * tr, tr)],
                    out_hbm.at[cid, pl.ds(sid * tr, tr)])


def _sc_degree(e3, n_pad_rows):
    rw = e3.shape[1] // NW
    f = pl.kernel(
        _deg_body,
        out_type=jax.ShapeDtypeStruct((NC, n_pad_rows, L), jnp.float32),
        mesh=_MESH,
        scratch_types=[
            pltpu.VMEM((rw, EB), jnp.int32),
            pltpu.VMEM((n_pad_rows, L), jnp.float32),
            pltpu.VMEM((n_pad_rows // EB, EB), jnp.int32),
            pltpu.VMEM_SHARED((n_pad_rows, L), jnp.float32),
        ],
        compiler_params=_SC_PARAMS,
    )
    return f(e3)


NBUF = 5


def _agg_body(g_hbm, e3_hbm, out_hbm,
              sidx_v, didx_v, rows_v, z_v, acc_sh, gsem, ssem):
    cid = lax.axis_index("c")
    sid = lax.axis_index("s")
    wid = sid * NC + cid
    rw = sidx_v.shape[0]
    d = rows_v.shape[2]
    zr = z_v.shape[0]
    nblk = rw // NBUF

    pltpu.sync_copy(e3_hbm.at[0, pl.ds(wid * rw, rw)], sidx_v)
    pltpu.sync_copy(e3_hbm.at[1, pl.ds(wid * rw, rw)], didx_v)

    zero16 = jnp.zeros((L,), jnp.float32)
    offs = _row_offsets(d)

    def zbody(i, carry):
        for off in offs:
            z_v[i, pl.ds(off, L)] = zero16
        return carry

    lax.fori_loop(0, zr, zbody, 0)
    pltpu.sync_copy(z_v, acc_sh.at[pl.ds(sid * zr, zr)])
    plsc.subcore_barrier()

    def gather(r, b):
        return pltpu.async_copy(g_hbm.at[sidx_v.at[r]], rows_v.at[b],
                                gsem.at[b])

    def scatter(r, b):
        return pltpu.async_copy(rows_v.at[b], acc_sh.at[didx_v.at[r]],
                                ssem.at[b], add=True)

    def wait_gather(r, b):
        pltpu.make_async_copy(g_hbm.at[sidx_v.at[r]], rows_v.at[b],
                              gsem.at[b]).wait()

    def wait_scatter(r, b):
        pltpu.make_async_copy(rows_v.at[b], acc_sh.at[didx_v.at[r]],
                              ssem.at[b]).wait()

    for b in range(NBUF):
        gather(b, b)

    def blk(i, carry):
        base = i * NBUF
        for b in range(NBUF):
            wait_gather(base + b, b)
            scatter(base + b, b)

        @pl.when(i < nblk - 1)
        def _():
            for b in range(NBUF):
                wait_scatter(base + b, b)
                gather(base + NBUF + b, b)

        @pl.when(i == nblk - 1)
        def _():
            for b in range(NBUF):
                wait_scatter(base + b, b)

        return carry

    lax.fori_loop(0, nblk, blk, 0)
    plsc.subcore_barrier()
    pltpu.sync_copy(acc_sh.at[pl.ds(sid * zr, zr)],
                    out_hbm.at[cid, pl.ds(sid * zr, zr)])


def _sc_aggregate(g, e3):
    n, d = g.shape
    rw = e3.shape[1] // NW
    f = pl.kernel(
        _agg_body,
        out_type=jax.ShapeDtypeStruct((NC, n, d), jnp.float32),
        mesh=_MESH,
        scratch_types=[
            pltpu.VMEM((rw, EB), jnp.int32),
            pltpu.VMEM((rw, EB), jnp.int32),
            pltpu.VMEM((NBUF, EB, d), jnp.float32),
            pltpu.VMEM((n // NS, d), jnp.float32),
            pltpu.VMEM_SHARED((n, d), jnp.float32),
            pltpu.SemaphoreType.DMA((NBUF,)),
            pltpu.SemaphoreType.DMA((NBUF,)),
        ],
        compiler_params=_SC_PARAMS,
    )
    return f(g, e3)


def _pack_rows(z, d):
    n = z.shape[0]
    lcm = d * 128 // math.gcd(d, 128)
    p, r = lcm // d, lcm // 128
    z3 = jnp.reshape(z, (n // p, p, d))
    rows = []
    for i in range(r):
        pieces, start = [], 128 * i
        while start < 128 * (i + 1):
            k, o = start // d, start % d
            end = min(d, o + 128 * (i + 1) - start)
            pieces.append(z3[:, k, o:end])
            start += end - o
        rows.append(jnp.concatenate(pieces, axis=1))
    st = jnp.stack(rows, axis=1)
    return jnp.reshape(st, (n * d // 128, 128))


def _unpack_rows(zq, n, d):
    lcm = d * 128 // math.gcd(d, 128)
    p, r = lcm // d, lcm // 128
    z3 = jnp.reshape(zq, (n // p, r, 128))
    nodes = []
    for k in range(p):
        pieces, start = [], d * k
        while start < d * (k + 1):
            i, o = start // 128, start % 128
            end = min(128, o + d * (k + 1) - start)
            pieces.append(z3[:, i, o:end])
            start += end - o
        nodes.append(jnp.concatenate(pieces, axis=1))
    st = jnp.stack(nodes, axis=1)
    return jnp.reshape(st, (n, d))


def _tc1_body(x_ref, w1_ref, dinv16_ref, g1p_ref):
    h = jnp.dot(x_ref[...], w1_ref[...], preferred_element_type=jnp.float32)
    g1p_ref[...] = _pack_rows(h, w1_ref.shape[1]) * dinv16_ref[...]


def _tc2_body(a_ref, g1p_ref, dinv16_ref, b1r_ref, up_ref):
    zp = dinv16_ref[...] * (a_ref[0] + a_ref[1] + g1p_ref[...]) + b1r_ref[...]
    up_ref[...] = dinv16_ref[...] * jnp.maximum(zp, 0.0)


def _tc3_body(a_ref, up_ref, dinv40_ref, w2_ref, b2_ref, o_ref):
    n, c = o_ref.shape
    hdim = w2_ref.shape[0]
    sp = a_ref[0] + a_ref[1] + up_ref[...]
    s = _unpack_rows(sp, n, hdim)
    h2 = jnp.dot(s, w2_ref[...], preferred_element_type=jnp.float32)
    z = _unpack_rows(dinv40_ref[...], n, c) * h2 + b2_ref[...]
    m = jnp.max(z, axis=1, keepdims=True)
    lse = jnp.log(jnp.sum(jnp.exp(z - m), axis=1, keepdims=True))
    o_ref[...] = z - m - lse


def _tc1(x, w1, dinv16p):
    return pl.pallas_call(
        _tc1_body,
        out_shape=jax.ShapeDtypeStruct(dinv16p.shape, jnp.float32),
    )(x, w1, dinv16p)


def _tc2(a1p, g1p, dinv16p, b1r):
    return pl.pallas_call(
        _tc2_body,
        out_shape=jax.ShapeDtypeStruct(dinv16p.shape, jnp.float32),
    )(a1p, g1p, dinv16p, b1r)


def _tc3(a2p, up, dinv40p, w2, b2, n):
    c = b2.shape[0]
    return pl.pallas_call(
        _tc3_body,
        out_shape=jax.ShapeDtypeStruct((n, c), jnp.float32),
    )(a2p, up, dinv40p, w2, b2)


def kernel(x, edge_index, W1, b1, W2, b2):
    n = x.shape[0]
    e = edge_index.shape[1]
    e3 = edge_index.reshape(2, e // EB, EB)
    n_rows = -(-n // (L * NS)) * NS

    hdim = W1.shape[1]
    c = W2.shape[1]

    dp = _sc_degree(e3, n_rows)
    deg = (dp[0] + dp[1]).reshape(n_rows * L)[:n] + 1.0
    dinv = lax.rsqrt(deg)
    dinv16p = jnp.repeat(dinv, hdim).reshape(n * hdim // 128, 128)
    dinv40p = jnp.repeat(dinv, c).reshape(n * c // 128, 128)
    b1r = jnp.tile(b1, 128 // hdim)

    g1p = _tc1(x, W1, dinv16p)
    g1 = g1p.reshape(n, hdim)
    a1 = _sc_aggregate(g1, e3)
    a1p = a1.reshape(NC, n * hdim // 128, 128)
    up = _tc2(a1p, g1p, dinv16p, b1r)
    a2 = _sc_aggregate(up.reshape(n, hdim), e3)
    a2p = a2.reshape(NC, n * hdim // 128, 128)
    return _tc3(a2p, up, dinv40p, W2, b2, n)


# speedup vs baseline: 64.2959x; 1.0167x over previous
import functools
import math

import jax
import jax.numpy as jnp
from jax import lax
from jax.experimental import pallas as pl
from jax.experimental.pallas import tpu as pltpu
from jax.experimental.pallas import tpu_sc as plsc

NC = 2
NS = 16
L = 16
NW = NC * NS
EB = 80

_MESH = plsc.VectorSubcoreMesh(core_axis_name="c", subcore_axis_name="s",
                               num_cores=NC, num_subcores=NS)
_SC_PARAMS = pltpu.CompilerParams(use_tc_tiling_on_sc=False,
                                  needs_layout_passes=False)


def _row_offsets(d):
    offs = list(range(0, d - L + 1, L))
    if d % L:
        offs.append(d - L)
    return offs


def _deg_body(e3_hbm, out_hbm, idx_v, hist_v, iidx_v, acc_sh):
    cid = lax.axis_index("c")
    sid = lax.axis_index("s")
    wid = sid * NC + cid
    rw = idx_v.shape[0]
    hr = hist_v.shape[0]
    tr = hr // NS

    pltpu.sync_copy(e3_hbm.at[1, pl.ds(wid * rw, rw)], idx_v)

    zero16 = jnp.zeros((L,), jnp.float32)
    iota16 = lax.iota(jnp.int32, L)

    def zbody(i, carry):
        hist_v[i, :] = zero16
        return carry

    lax.fori_loop(0, hr, zbody, 0)

    for b in range(hr // EB):
        for c in range(EB // L):
            iidx_v[b, pl.ds(c * L, L)] = b * EB + c * L + iota16
    pltpu.sync_copy(hist_v.at[pl.ds(sid * tr, tr)],
                    acc_sh.at[pl.ds(sid * tr, tr)])
    plsc.subcore_barrier()

    one16 = jnp.ones((L,), jnp.float32)
    m15 = jnp.full((L,), L - 1, jnp.int32)

    def body(r, carry):
        for c in range(EB // L):
            idx = idx_v[r, pl.ds(c * L, L)]
            row = lax.shift_right_logical(idx, 4)
            col = jnp.bitwise_and(idx, m15)
            plsc.addupdate_scatter(hist_v, [row, col], one16)
        return carry

    lax.fori_loop(0, rw, body, 0)

    for b in range(hr // EB):
        pltpu.sync_copy(hist_v.at[pl.ds(b * EB, EB)],
                        acc_sh.at[iidx_v.at[b]], add=True)
    plsc.subcore_barrier()
    pltpu.sync_copy(acc_sh.at[pl.ds(sid * tr, tr)],
                    out_hbm.at[cid, pl.ds(sid * tr, tr)])


def _sc_degree(e3, n_pad_rows):
    rw = e3.shape[1] // NW
    f = pl.kernel(
        _deg_body,
        out_type=jax.ShapeDtypeStruct((NC, n_pad_rows, L), jnp.float32),
        mesh=_MESH,
        scratch_types=[
            pltpu.VMEM((rw, EB), jnp.int32),
            pltpu.VMEM((n_pad_rows, L), jnp.float32),
            pltpu.VMEM((n_pad_rows // EB, EB), jnp.int32),
            pltpu.VMEM_SHARED((n_pad_rows, L), jnp.float32),
        ],
        compiler_params=_SC_PARAMS,
    )
    return f(e3)


NBUF = 5


def _agg_body(g_hbm, e3_hbm, out_hbm,
              sidx_v, didx_v, rows_v, z_v, acc_sh, gsem, ssem):
    cid = lax.axis_index("c")
    sid = lax.axis_index("s")
    wid = sid * NC + cid
    rw = sidx_v.shape[0]
    d = rows_v.shape[2]
    zr = z_v.shape[0]
    nblk = rw // NBUF

    pltpu.sync_copy(e3_hbm.at[0, pl.ds(wid * rw, rw)], sidx_v)
    pltpu.sync_copy(e3_hbm.at[1, pl.ds(wid * rw, rw)], didx_v)

    zero16 = jnp.zeros((L,), jnp.float32)
    offs = _row_offsets(d)

    def zbody(i, carry):
        for off in offs:
            z_v[i, pl.ds(off, L)] = zero16
        return carry

    lax.fori_loop(0, zr, zbody, 0)
    pltpu.sync_copy(z_v, acc_sh.at[pl.ds(sid * zr, zr)])
    plsc.subcore_barrier()

    def gather(r, b):
        return pltpu.async_copy(g_hbm.at[sidx_v.at[r]], rows_v.at[b],
                                gsem.at[b])

    def scatter(r, b):
        return pltpu.async_copy(rows_v.at[b], acc_sh.at[didx_v.at[r]],
                                ssem.at[b], add=True)

    def wait_gather(r, b):
        pltpu.make_async_copy(g_hbm.at[sidx_v.at[r]], rows_v.at[b],
                              gsem.at[b]).wait()

    def wait_scatter(r, b):
        pltpu.make_async_copy(rows_v.at[b], acc_sh.at[didx_v.at[r]],
                              ssem.at[b]).wait()

    for b in range(NBUF):
        gather(b, b)

    def blk(i, carry):
        base = i * NBUF
        for b in range(NBUF):
            wait_gather(base + b, b)
            scatter(base + b, b)

        @pl.when(i < nblk - 1)
        def _():
            for b in range(NBUF):
                wait_scatter(base + b, b)
                gather(base + NBUF + b, b)

        @pl.when(i == nblk - 1)
        def _():
            for b in range(NBUF):
                wait_scatter(base + b, b)

        return carry

    lax.fori_loop(0, nblk, blk, 0)
    plsc.subcore_barrier()
    pltpu.sync_copy(acc_sh.at[pl.ds(sid * zr, zr)],
                    out_hbm.at[cid, pl.ds(sid * zr, zr)])


def _sc_aggregate(g, e3):
    n, d = g.shape
    rw = e3.shape[1] // NW
    f = pl.kernel(
        _agg_body,
        out_type=jax.ShapeDtypeStruct((NC, n, d), jnp.float32),
        mesh=_MESH,
        scratch_types=[
            pltpu.VMEM((rw, EB), jnp.int32),
            pltpu.VMEM((rw, EB), jnp.int32),
            pltpu.VMEM((NBUF, EB, d), jnp.float32),
            pltpu.VMEM((n // NS, d), jnp.float32),
            pltpu.VMEM_SHARED((n, d), jnp.float32),
            pltpu.SemaphoreType.DMA((NBUF,)),
            pltpu.SemaphoreType.DMA((NBUF,)),
        ],
        compiler_params=_SC_PARAMS,
    )
    return f(g, e3)


def _pack_rows(z, d):
    n = z.shape[0]
    lcm = d * 128 // math.gcd(d, 128)
    p, r = lcm // d, lcm // 128
    z3 = jnp.reshape(z, (n // p, p, d))
    rows = []
    for i in range(r):
        pieces, start = [], 128 * i
        while start < 128 * (i + 1):
            k, o = start // d, start % d
            end = min(d, o + 128 * (i + 1) - start)
            pieces.append(z3[:, k, o:end])
            start += end - o
        rows.append(jnp.concatenate(pieces, axis=1))
    st = jnp.stack(rows, axis=1)
    return jnp.reshape(st, (n * d // 128, 128))


def _unpack_rows(zq, n, d):
    lcm = d * 128 // math.gcd(d, 128)
    p, r = lcm // d, lcm // 128
    z3 = jnp.reshape(zq, (n // p, r, 128))
    nodes = []
    for k in range(p):
        pieces, start = [], d * k
        while start < d * (k + 1):
            i, o = start // 128, start % 128
            end = min(128, o + d * (k + 1) - start)
            pieces.append(z3[:, i, o:end])
            start += end - o
        nodes.append(jnp.concatenate(pieces, axis=1))
    st = jnp.stack(nodes, axis=1)
    return jnp.reshape(st, (n, d))


def _expand_dinv(dinv, d):
    nr = dinv.shape[0]
    i_idx = lax.broadcasted_iota(jnp.int32, (L, L * d), 0)
    j_idx = lax.broadcasted_iota(jnp.int32, (L, L * d), 1)
    sel = (i_idx == j_idx // d).astype(jnp.float32)
    o = jnp.dot(dinv, sel, preferred_element_type=jnp.float32)
    r = L * d // 128
    st = jnp.stack([o[:, 128 * m:128 * (m + 1)] for m in range(r)], axis=1)
    return jnp.reshape(st, (nr * r, 128))


def _tc1_body(x_ref, w1_ref, dp_ref, g1p_ref, dinv16_ref, dinv40_ref, c):
    deg = dp_ref[0] + dp_ref[1] + 1.0
    dinv = lax.rsqrt(deg)
    hdim = w1_ref.shape[1]
    d16 = _expand_dinv(dinv, hdim)[:g1p_ref.shape[0]]
    dinv16_ref[...] = d16
    dinv40_ref[...] = _expand_dinv(dinv, c)[:dinv40_ref.shape[0]]
    h = jnp.dot(x_ref[...], w1_ref[...], preferred_element_type=jnp.float32)
    g1p_ref[...] = _pack_rows(h, hdim) * d16


def _tc2_body(a_ref, g1p_ref, dinv16_ref, b1r_ref, up_ref):
    zp = dinv16_ref[...] * (a_ref[0] + a_ref[1] + g1p_ref[...]) + b1r_ref[...]
    up_ref[...] = dinv16_ref[...] * jnp.maximum(zp, 0.0)


def _tc3_body(a_ref, up_ref, dinv40_ref, w2_ref, b2_ref, o_ref):
    n, c = o_ref.shape
    hdim = w2_ref.shape[0]
    sp = a_ref[0] + a_ref[1] + up_ref[...]
    s = _unpack_rows(sp, n, hdim)
    h2 = jnp.dot(s, w2_ref[...], preferred_element_type=jnp.float32)
    z = _unpack_rows(dinv40_ref[...], n, c) * h2 + b2_ref[...]
    m = jnp.max(z, axis=1, keepdims=True)
    lse = jnp.log(jnp.sum(jnp.exp(z - m), axis=1, keepdims=True))
    o_ref[...] = z - m - lse


def _tc1(x, w1, dp, c):
    n = x.shape[0]
    hdim = w1.shape[1]
    return pl.pallas_call(
        functools.partial(_tc1_body, c=c),
        out_shape=(
            jax.ShapeDtypeStruct((n * hdim // 128, 128), jnp.float32),
            jax.ShapeDtypeStruct((n * hdim // 128, 128), jnp.float32),
            jax.ShapeDtypeStruct((n * c // 128, 128), jnp.float32),
        ),
    )(x, w1, dp)


def _tc2(a1p, g1p, dinv16p, b1r):
    return pl.pallas_call(
        _tc2_body,
        out_shape=jax.ShapeDtypeStruct(dinv16p.shape, jnp.float32),
    )(a1p, g1p, dinv16p, b1r)


def _tc3(a2p, up, dinv40p, w2, b2, n):
    c = b2.shape[0]
    return pl.pallas_call(
        _tc3_body,
        out_shape=jax.ShapeDtypeStruct((n, c), jnp.float32),
    )(a2p, up, dinv40p, w2, b2)


def kernel(x, edge_index, W1, b1, W2, b2):
    n = x.shape[0]
    e = edge_index.shape[1]
    e3 = edge_index.reshape(2, e // EB, EB)
    n_rows = -(-n // (L * NS)) * NS

    hdim = W1.shape[1]
    c = W2.shape[1]

    dp = _sc_degree(e3, n_rows)
    b1r = jnp.tile(b1, 128 // hdim)

    g1p, dinv16p, dinv40p = _tc1(x, W1, dp, c)
    g1 = g1p.reshape(n, hdim)
    a1 = _sc_aggregate(g1, e3)
    a1p = a1.reshape(NC, n * hdim // 128, 128)
    up = _tc2(a1p, g1p, dinv16p, b1r)
    a2 = _sc_aggregate(up.reshape(n, hdim), e3)
    a2p = a2.reshape(NC, n * hdim // 128, 128)
    return _tc3(a2p, up, dinv40p, W2, b2, n)


# speedup vs baseline: 65.7555x; 1.0227x over previous
import functools
import math

import jax
import jax.numpy as jnp
from jax import lax
from jax.experimental import pallas as pl
from jax.experimental.pallas import tpu as pltpu
from jax.experimental.pallas import tpu_sc as plsc

NC = 2
NS = 16
L = 16
NW = NC * NS
EB = 80

_MESH = plsc.VectorSubcoreMesh(core_axis_name="c", subcore_axis_name="s",
                               num_cores=NC, num_subcores=NS)
_SC_PARAMS = pltpu.CompilerParams(use_tc_tiling_on_sc=False,
                                  needs_layout_passes=False)


def _row_offsets(d):
    offs = list(range(0, d - L + 1, L))
    if d % L:
        offs.append(d - L)
    return offs


def _deg_body(e3_hbm, out_hbm, idx_v, hist_v, iidx_v, acc_sh):
    cid = lax.axis_index("c")
    sid = lax.axis_index("s")
    wid = sid * NC + cid
    rw = idx_v.shape[0]
    hr = hist_v.shape[0]
    tr = hr // NS

    pltpu.sync_copy(e3_hbm.at[1, pl.ds(wid * rw, rw)], idx_v)

    zero16 = jnp.zeros((L,), jnp.float32)
    iota16 = lax.iota(jnp.int32, L)

    def zbody(i, carry):
        hist_v[i, :] = zero16
        return carry

    lax.fori_loop(0, hr, zbody, 0)

    for b in range(hr // EB):
        for c in range(EB // L):
            iidx_v[b, pl.ds(c * L, L)] = b * EB + c * L + iota16
    pltpu.sync_copy(hist_v.at[pl.ds(sid * tr, tr)],
                    acc_sh.at[pl.ds(sid * tr, tr)])
    plsc.subcore_barrier()

    one16 = jnp.ones((L,), jnp.float32)
    m15 = jnp.full((L,), L - 1, jnp.int32)

    def body(r, carry):
        for c in range(EB // L):
            idx = idx_v[r, pl.ds(c * L, L)]
            row = lax.shift_right_logical(idx, 4)
            col = jnp.bitwise_and(idx, m15)
            plsc.addupdate_scatter(hist_v, [row, col], one16)
        return carry

    lax.fori_loop(0, rw, body, 0)

    for b in range(hr // EB):
        pltpu.sync_copy(hist_v.at[pl.ds(b * EB, EB)],
                        acc_sh.at[iidx_v.at[b]], add=True)
    plsc.subcore_barrier()
    pltpu.sync_copy(acc_sh.at[pl.ds(sid * tr, tr)],
                    out_hbm.at[cid, pl.ds(sid * tr, tr)])


def _sc_degree(e3, n_pad_rows):
    rw = e3.shape[1] // NW
    f = pl.kernel(
        _deg_body,
        out_type=jax.ShapeDtypeStruct((NC, n_pad_rows, L), jnp.float32),
        mesh=_MESH,
        scratch_types=[
            pltpu.VMEM((rw, EB), jnp.int32),
            pltpu.VMEM((n_pad_rows, L), jnp.float32),
            pltpu.VMEM((n_pad_rows // EB, EB), jnp.int32),
            pltpu.VMEM_SHARED((n_pad_rows, L), jnp.float32),
        ],
        compiler_params=_SC_PARAMS,
    )
    return f(e3)


NBUF = 5


def _agg_body(g_hbm, e3_hbm, out_hbm,
              sidx_v, didx_v, rows_v, z_v, acc_sh, gsem, ssem):
    cid = lax.axis_index("c")
    sid = lax.axis_index("s")
    wid = sid * NC + cid
    rw = sidx_v.shape[0]
    d = rows_v.shape[2]
    zr = z_v.shape[0]
    nblk = rw // NBUF

    pltpu.sync_copy(e3_hbm.at[0, pl.ds(wid * rw, rw)], sidx_v)
    pltpu.sync_copy(e3_hbm.at[1, pl.ds(wid * rw, rw)], didx_v)

    zero16 = jnp.zeros((L,), jnp.float32)
    offs = _row_offsets(d)

    def zbody(i, carry):
        for off in offs:
            z_v[i, pl.ds(off, L)] = zero16
        return carry

    lax.fori_loop(0, zr, zbody, 0)
    pltpu.sync_copy(z_v, acc_sh.at[pl.ds(sid * zr, zr)])
    plsc.subcore_barrier()

    def gather(r, b):
        return pltpu.async_copy(g_hbm.at[sidx_v.at[r]], rows_v.at[b],
                                gsem.at[b])

    def scatter(r, b):
        return pltpu.async_copy(rows_v.at[b], acc_sh.at[didx_v.at[r]],
                                ssem.at[b], add=True)

    def wait_gather(r, b):
        pltpu.make_async_copy(g_hbm.at[sidx_v.at[r]], rows_v.at[b],
                              gsem.at[b]).wait()

    def wait_scatter(r, b):
        pltpu.make_async_copy(rows_v.at[b], acc_sh.at[didx_v.at[r]],
                              ssem.at[b]).wait()

    for b in range(NBUF):
        gather(b, b)

    def blk(i, carry):
        base = i * NBUF
        for b in range(NBUF):
            wait_gather(base + b, b)
            scatter(base + b, b)

        @pl.when(i < nblk - 1)
        def _():
            for b in range(NBUF):
                wait_scatter(base + b, b)
                gather(base + NBUF + b, b)

        @pl.when(i == nblk - 1)
        def _():
            for b in range(NBUF):
                wait_scatter(base + b, b)

        return carry

    lax.fori_loop(0, nblk, blk, 0)
    plsc.subcore_barrier()
    pltpu.sync_copy(acc_sh.at[pl.ds(sid * zr, zr)],
                    out_hbm.at[cid, pl.ds(sid * zr, zr)])


def _sc_aggregate(g, e3):
    n, d = g.shape
    rw = e3.shape[1] // NW
    f = pl.kernel(
        _agg_body,
        out_type=jax.ShapeDtypeStruct((NC, n, d), jnp.float32),
        mesh=_MESH,
        scratch_types=[
            pltpu.VMEM((rw, EB), jnp.int32),
            pltpu.VMEM((rw, EB), jnp.int32),
            pltpu.VMEM((NBUF, EB, d), jnp.float32),
            pltpu.VMEM((n // NS, d), jnp.float32),
            pltpu.VMEM_SHARED((n, d), jnp.float32),
            pltpu.SemaphoreType.DMA((NBUF,)),
            pltpu.SemaphoreType.DMA((NBUF,)),
        ],
        compiler_params=_SC_PARAMS,
    )
    return f(g, e3)


def _pack_rows(z, d):
    n = z.shape[0]
    lcm = d * 128 // math.gcd(d, 128)
    p, r = lcm // d, lcm // 128
    z3 = jnp.reshape(z, (n // p, p, d))
    rows = []
    for i in range(r):
        pieces, start = [], 128 * i
        while start < 128 * (i + 1):
            k, o = start // d, start % d
            end = min(d, o + 128 * (i + 1) - start)
            pieces.append(z3[:, k, o:end])
            start += end - o
        rows.append(jnp.concatenate(pieces, axis=1))
    st = jnp.stack(rows, axis=1)
    return jnp.reshape(st, (n * d // 128, 128))


def _unpack_rows(zq, n, d):
    lcm = d * 128 // math.gcd(d, 128)
    p, r = lcm // d, lcm // 128
    z3 = jnp.reshape(zq, (n // p, r, 128))
    nodes = []
    for k in range(p):
        pieces, start = [], d * k
        while start < d * (k + 1):
            i, o = start // 128, start % 128
            end = min(128, o + d * (k + 1) - start)
            pieces.append(z3[:, i, o:end])
            start += end - o
        nodes.append(jnp.concatenate(pieces, axis=1))
    st = jnp.stack(nodes, axis=1)
    return jnp.reshape(st, (n, d))


def _expand_dinv(dinv, d):
    nr = dinv.shape[0]
    i_idx = lax.broadcasted_iota(jnp.int32, (L, L * d), 0)
    j_idx = lax.broadcasted_iota(jnp.int32, (L, L * d), 1)
    sel = (i_idx == j_idx // d).astype(jnp.float32)
    o = jnp.dot(dinv, sel, preferred_element_type=jnp.float32)
    r = L * d // 128
    st = jnp.stack([o[:, 128 * m:128 * (m + 1)] for m in range(r)], axis=1)
    return jnp.reshape(st, (nr * r, 128))


def _tc0_body(x_ref, w1_ref, h1p_ref):
    h = jnp.dot(x_ref[...], w1_ref[...], preferred_element_type=jnp.float32)
    h1p_ref[...] = _pack_rows(h, w1_ref.shape[1])


def _tc1_body(dp_ref, h1p_ref, g1p_ref, dinv16_ref, dinv40_ref, hdim, c):
    deg = dp_ref[0] + dp_ref[1] + 1.0
    dinv = lax.rsqrt(deg)
    d16 = _expand_dinv(dinv, hdim)[:g1p_ref.shape[0]]
    dinv16_ref[...] = d16
    dinv40_ref[...] = _expand_dinv(dinv, c)[:dinv40_ref.shape[0]]
    g1p_ref[...] = h1p_ref[...] * d16


def _tc2_body(a_ref, g1p_ref, dinv16_ref, b1r_ref, up_ref):
    zp = dinv16_ref[...] * (a_ref[0] + a_ref[1] + g1p_ref[...]) + b1r_ref[...]
    up_ref[...] = dinv16_ref[...] * jnp.maximum(zp, 0.0)


def _tc3_body(a_ref, up_ref, dinv40_ref, w2_ref, b2_ref, o_ref):
    n, c = o_ref.shape
    hdim = w2_ref.shape[0]
    sp = a_ref[0] + a_ref[1] + up_ref[...]
    s = _unpack_rows(sp, n, hdim)
    h2 = jnp.dot(s, w2_ref[...], preferred_element_type=jnp.float32)
    z = _unpack_rows(dinv40_ref[...], n, c) * h2 + b2_ref[...]
    m = jnp.max(z, axis=1, keepdims=True)
    lse = jnp.log(jnp.sum(jnp.exp(z - m), axis=1, keepdims=True))
    o_ref[...] = z - m - lse


def _tc0(x, w1):
    n = x.shape[0]
    hdim = w1.shape[1]
    return pl.pallas_call(
        _tc0_body,
        out_shape=jax.ShapeDtypeStruct((n * hdim // 128, 128), jnp.float32),
    )(x, w1)


def _tc1(dp, h1p, n, hdim, c):
    return pl.pallas_call(
        functools.partial(_tc1_body, hdim=hdim, c=c),
        out_shape=(
            jax.ShapeDtypeStruct((n * hdim // 128, 128), jnp.float32),
            jax.ShapeDtypeStruct((n * hdim // 128, 128), jnp.float32),
            jax.ShapeDtypeStruct((n * c // 128, 128), jnp.float32),
        ),
    )(dp, h1p)


def _tc2(a1p, g1p, dinv16p, b1r):
    return pl.pallas_call(
        _tc2_body,
        out_shape=jax.ShapeDtypeStruct(dinv16p.shape, jnp.float32),
    )(a1p, g1p, dinv16p, b1r)


def _tc3(a2p, up, dinv40p, w2, b2, n):
    c = b2.shape[0]
    return pl.pallas_call(
        _tc3_body,
        out_shape=jax.ShapeDtypeStruct((n, c), jnp.float32),
    )(a2p, up, dinv40p, w2, b2)


def kernel(x, edge_index, W1, b1, W2, b2):
    n = x.shape[0]
    e = edge_index.shape[1]
    e3 = edge_index.reshape(2, e // EB, EB)
    n_rows = -(-n // (L * NS)) * NS

    hdim = W1.shape[1]
    c = W2.shape[1]

    h1p = _tc0(x, W1)
    dp = _sc_degree(e3, n_rows)
    b1r = jnp.tile(b1, 128 // hdim)

    g1p, dinv16p, dinv40p = _tc1(dp, h1p, n, hdim, c)
    g1 = g1p.reshape(n, hdim)
    a1 = _sc_aggregate(g1, e3)
    a1p = a1.reshape(NC, n * hdim // 128, 128)
    up = _tc2(a1p, g1p, dinv16p, b1r)
    a2 = _sc_aggregate(up.reshape(n, hdim), e3)
    a2p = a2.reshape(NC, n * hdim // 128, 128)
    return _tc3(a2p, up, dinv40p, W2, b2, n)
